# baseline (speedup 1.0000x reference)
import functools

import numpy as np

import jax
import jax.numpy as jnp
from jax import lax
from jax.experimental import pallas as pl
from jax.experimental.pallas import tpu as pltpu
from jax.experimental.pallas import tpu_sc as plsc

EMB_K = 1024
EMB_D = 64
BT = 2048

SC_NC = 2
SC_NS = 16
SC_NW = SC_NC * SC_NS
SC_CH = 128


def _one_argmin(x, emb2, cn, iotf):
    m2 = jnp.dot(emb2, x, preferred_element_type=jnp.float32)
    rn = jnp.sum(x * x, axis=0, keepdims=True)
    dist = (cn - m2) + rn
    mn = jnp.min(dist, axis=0, keepdims=True)
    idxf = jnp.min(jnp.where(dist == mn, iotf, 2.0 * EMB_K), axis=0)
    return idxf.astype(jnp.int32)


def _argmin_body(x1_ref, x0_ref, e1_ref, e0_ref, cn1_ref, cn0_ref,
                 iotf_ref, o1_ref, o0_ref, x1b, x0b, sem):
    b = pl.program_id(0)
    nb = pl.num_programs(0)
    slot = lax.rem(b, 2)
    nslot = lax.rem(b + 1, 2)

    def start(bi, si):
        pltpu.make_async_copy(x1_ref.at[bi], x1b.at[si], sem.at[si, 0]).start()
        pltpu.make_async_copy(x0_ref.at[bi], x0b.at[si], sem.at[si, 1]).start()

    def wait(bi, si):
        pltpu.make_async_copy(x1_ref.at[bi], x1b.at[si], sem.at[si, 0]).wait()
        pltpu.make_async_copy(x0_ref.at[bi], x0b.at[si], sem.at[si, 1]).wait()

    @pl.when(b == 0)
    def _():
        start(b, slot)
    wait(b, slot)

    @pl.when(b + 1 < nb)
    def _():
        start(b + 1, nslot)

    iotf = iotf_ref[...]
    o1_ref[0, 0, :] = _one_argmin(x1b[slot], e1_ref[...], cn1_ref[...], iotf)
    o0_ref[0, 0, :] = _one_argmin(x0b[slot], e0_ref[...], cn0_ref[...], iotf)


def _tc_argmin2(enct1, enct0, emb2_1, emb2_0, cn1, cn0):
    nb, _, nt = enct1.shape[0], enct1.shape[1], enct1.shape[2]
    nj = nt // BT
    xspec = pl.BlockSpec(memory_space=pl.ANY)
    espec = pl.BlockSpec((EMB_K, EMB_D), lambda b, j: (0, 0))
    cspec = pl.BlockSpec((EMB_K, 1), lambda b, j: (0, 0))
    ospec = pl.BlockSpec((1, 1, BT), lambda b, j: (b, 0, j))
    oshape = jax.ShapeDtypeStruct((nb, 1, nt), jnp.int32)
    o1, o0 = pl.pallas_call(
        _argmin_body,
        grid=(nb, nj),
        in_specs=[xspec, xspec, espec, espec, cspec, cspec, cspec],
        out_specs=[ospec, ospec],
        out_shape=[oshape, oshape],
        scratch_shapes=[
            pltpu.VMEM((2, EMB_D, BT), jnp.float32),
            pltpu.VMEM((2, EMB_D, BT), jnp.float32),
            pltpu.SemaphoreType.DMA((2, 2)),
        ],
    )(enct1, enct0, emb2_1, emb2_0, cn1, cn0,
      jnp.asarray(np.arange(EMB_K, dtype=np.float32)[:, None]))
    return o1.reshape(-1), o0.reshape(-1)


def _sc_gather(table1, table0, idx1, idx0):
    n = idx1.shape[0]
    bw = n // SC_NW
    nch = bw // SC_CH
    mesh = plsc.VectorSubcoreMesh(core_axis_name="c", subcore_axis_name="s")

    @functools.partial(
        pl.kernel,
        mesh=mesh,
        out_type=jax.ShapeDtypeStruct((n, 2 * EMB_D), jnp.float32),
        scratch_types=[
            pltpu.VMEM((bw,), jnp.int32),
            pltpu.VMEM((bw,), jnp.int32),
            pltpu.VMEM((SC_CH, 2 * EMB_D), jnp.float32),
            pltpu.VMEM((SC_CH, 2 * EMB_D), jnp.float32),
            pltpu.VMEM_SHARED((EMB_K, 2 * EMB_D), jnp.float32),
            pltpu.VMEM_SHARED((EMB_K, 2 * EMB_D), jnp.float32),
            pltpu.SemaphoreType.DMA,
            pltpu.SemaphoreType.DMA,
            pltpu.SemaphoreType.DMA,
        ],
    )
    def gather_kernel(t1_hbm, t0_hbm, i1_hbm, i0_hbm, out_hbm,
                      i1_v, i0_v, ba_v, bb_v, t1_sp, t0_sp,
                      sem_g1, sem_g0, sem_w):
        sid = lax.axis_index("s")
        wid = sid * SC_NC + lax.axis_index("c")
        base = wid * bw
        bufs = (ba_v, bb_v)
        @pl.when(sid == 0)
        def _():
            pltpu.sync_copy(t1_hbm, t1_sp)
            pltpu.sync_copy(t0_hbm, t0_sp)
        plsc.subcore_barrier()
        pltpu.sync_copy(i1_hbm.at[pl.ds(base, bw)], i1_v)
        pltpu.sync_copy(i0_hbm.at[pl.ds(base, bw)], i0_v)

        def g1_start(c):
            return pltpu.async_copy(
                t1_sp.at[i1_v.at[pl.ds(c * SC_CH, SC_CH)]],
                bufs[c % 2], sem_g1)

        g1 = g1_start(0)
        writes = [None] * nch
        for c in range(nch):
            b_v = bufs[c % 2]
            g1.wait()
            g0 = pltpu.async_copy(
                t0_sp.at[i0_v.at[pl.ds(c * SC_CH, SC_CH)]],
                b_v, sem_g0, add=True)
            if c + 1 < nch:
                if c >= 1:
                    writes[c - 1].wait()
                g1 = g1_start(c + 1)
            g0.wait()
            writes[c] = pltpu.async_copy(
                b_v, out_hbm.at[pl.ds(base + c * SC_CH, SC_CH)], sem_w)
        writes[nch - 2].wait()
        writes[nch - 1].wait()

    return gather_kernel(table1, table0, idx1, idx0)


def kernel(enc0, enc1, codebook0, codebook1):
    b, t, d = enc0.shape
    f0 = enc0.reshape(-1, d)
    f1 = enc1.reshape(-1, d)
    cn0 = jnp.sum(codebook0 ** 2, axis=1)[:, None]
    cn1 = jnp.sum(codebook1 ** 2, axis=1)[:, None]
    idx1, idx0 = _tc_argmin2(
        jnp.transpose(enc1, (0, 2, 1)), jnp.transpose(enc0, (0, 2, 1)),
        codebook1 * 2.0, codebook0 * 2.0, cn1, cn0)
    table1 = jnp.pad(codebook1, ((0, 0), (0, EMB_D)))
    table0 = jnp.pad(codebook0, ((0, 0), (EMB_D, 0)))
    rows = _sc_gather(table1, table0, idx1, idx0)
    return rows.reshape(b, t, 2 * d)


# speedup vs baseline: 1.0151x; 1.0151x over previous
import functools

import numpy as np

import jax
import jax.numpy as jnp
from jax import lax
from jax.experimental import pallas as pl
from jax.experimental.pallas import tpu as pltpu
from jax.experimental.pallas import tpu_sc as plsc

EMB_K = 1024
EMB_D = 64
BT = 2048

SC_NC = 2
SC_NS = 16
SC_NW = SC_NC * SC_NS
SC_CH = 128


def _one_argmin(x, emb2, cn, iotf):
    m2 = jnp.dot(emb2, x, preferred_element_type=jnp.float32)
    rn = jnp.sum(x * x, axis=0, keepdims=True)
    dist = (cn - m2) + rn
    mn = jnp.min(dist, axis=0, keepdims=True)
    idxf = jnp.min(jnp.where(dist == mn, iotf, 2.0 * EMB_K), axis=0)
    return idxf.astype(jnp.int32)


def _argmin_body(x1_ref, x0_ref, e1_ref, e0_ref, cn1_ref, cn0_ref,
                 iotf_ref, o1_ref, o0_ref):
    iotf = iotf_ref[...]
    o1_ref[0, 0, :] = _one_argmin(x1_ref[0], e1_ref[...], cn1_ref[...], iotf)
    o0_ref[0, 0, :] = _one_argmin(x0_ref[0], e0_ref[...], cn0_ref[...], iotf)


def _tc_argmin2(enct1, enct0, emb2_1, emb2_0, cn1, cn0):
    nb, _, nt = enct1.shape[0], enct1.shape[1], enct1.shape[2]
    nj = nt // BT
    xspec = pl.BlockSpec((1, EMB_D, BT), lambda b, j: (b, 0, j))
    espec = pl.BlockSpec((EMB_K, EMB_D), lambda b, j: (0, 0))
    cspec = pl.BlockSpec((EMB_K, 1), lambda b, j: (0, 0))
    ospec = pl.BlockSpec((1, 1, BT), lambda b, j: (b, 0, j))
    oshape = jax.ShapeDtypeStruct((nb, 1, nt), jnp.int32)
    o1, o0 = pl.pallas_call(
        _argmin_body,
        grid=(nb, nj),
        in_specs=[xspec, xspec, espec, espec, cspec, cspec, cspec],
        out_specs=[ospec, ospec],
        out_shape=[oshape, oshape],
    )(enct1, enct0, emb2_1, emb2_0, cn1, cn0,
      jnp.asarray(np.arange(EMB_K, dtype=np.float32)[:, None]))
    return o1.reshape(-1), o0.reshape(-1)


def _sc_gather(table1, table0, idx1, idx0):
    n = idx1.shape[0]
    bw = n // SC_NW
    nch = bw // SC_CH
    mesh = plsc.VectorSubcoreMesh(core_axis_name="c", subcore_axis_name="s")

    @functools.partial(
        pl.kernel,
        mesh=mesh,
        out_type=jax.ShapeDtypeStruct((n, 2 * EMB_D), jnp.float32),
        scratch_types=[
            pltpu.VMEM((bw,), jnp.int32),
            pltpu.VMEM((bw,), jnp.int32),
            pltpu.VMEM((SC_CH, 2 * EMB_D), jnp.float32),
            pltpu.VMEM((SC_CH, 2 * EMB_D), jnp.float32),
            pltpu.VMEM((SC_CH, 2 * EMB_D), jnp.float32),
            pltpu.VMEM((SC_CH, 2 * EMB_D), jnp.float32),
            pltpu.VMEM_SHARED((EMB_K, 2 * EMB_D), jnp.float32),
            pltpu.VMEM_SHARED((EMB_K, 2 * EMB_D), jnp.float32),
            pltpu.SemaphoreType.DMA,
            pltpu.SemaphoreType.DMA,
            pltpu.SemaphoreType.DMA,
        ],
    )
    def gather_kernel(t1_hbm, t0_hbm, i1_hbm, i0_hbm, out_hbm,
                      i1_v, i0_v, ba_v, bb_v, bc_v, bd_v, t1_sp, t0_sp,
                      sem_g1, sem_g0, sem_w):
        sid = lax.axis_index("s")
        wid = sid * SC_NC + lax.axis_index("c")
        base = wid * bw
        bufs = (ba_v, bb_v, bc_v, bd_v)
        @pl.when(sid == 0)
        def _():
            pltpu.sync_copy(t1_hbm, t1_sp)
            pltpu.sync_copy(t0_hbm, t0_sp)
        plsc.subcore_barrier()
        pltpu.sync_copy(i1_hbm.at[pl.ds(base, bw)], i1_v)
        pltpu.sync_copy(i0_hbm.at[pl.ds(base, bw)], i0_v)

        def g1_start(c):
            return pltpu.async_copy(
                t1_sp.at[i1_v.at[pl.ds(c * SC_CH, SC_CH)]],
                bufs[c % 4], sem_g1)

        g1d = [None] * nch
        writes = [None] * nch
        for c in range(min(3, nch)):
            g1d[c] = g1_start(c)
        for c in range(nch):
            b_v = bufs[c % 4]
            g1d[c].wait()
            g0 = pltpu.async_copy(
                t0_sp.at[i0_v.at[pl.ds(c * SC_CH, SC_CH)]],
                b_v, sem_g0, add=True)
            if c + 3 < nch:
                if c >= 1:
                    writes[c - 1].wait()
                g1d[c + 3] = g1_start(c + 3)
            g0.wait()
            writes[c] = pltpu.async_copy(
                b_v, out_hbm.at[pl.ds(base + c * SC_CH, SC_CH)], sem_w)
        for c in range(max(0, nch - 4), nch):
            writes[c].wait()

    return gather_kernel(table1, table0, idx1, idx0)


def kernel(enc0, enc1, codebook0, codebook1):
    b, t, d = enc0.shape
    f0 = enc0.reshape(-1, d)
    f1 = enc1.reshape(-1, d)
    cn0 = jnp.sum(codebook0 ** 2, axis=1)[:, None]
    cn1 = jnp.sum(codebook1 ** 2, axis=1)[:, None]
    idx1, idx0 = _tc_argmin2(
        jnp.transpose(enc1, (0, 2, 1)), jnp.transpose(enc0, (0, 2, 1)),
        codebook1 * 2.0, codebook0 * 2.0, cn1, cn0)
    table1 = jnp.pad(codebook1, ((0, 0), (0, EMB_D)))
    table0 = jnp.pad(codebook0, ((0, 0), (EMB_D, 0)))
    rows = _sc_gather(table1, table0, idx1, idx0)
    return rows.reshape(b, t, 2 * d)
